# baseline (speedup 1.0000x reference)
import functools

import jax
import jax.numpy as jnp
from jax import lax
from jax.experimental import pallas as pl
from jax.experimental.pallas import tpu as pltpu
from jax.experimental.pallas import tpu_sc as plsc

B = 16384
NTOK = 8 * B
NW = 32
TPW = NTOK // NW
L = 16


def _table_body(aeT, peT, wa1T, ba1, wa2T, ba2, wp1T, bp1, wp2T, bp2, out_ref):
    ba1c = jnp.transpose(ba1[...], (1, 0))
    ba2c = jnp.transpose(ba2[...], (1, 0))
    bp1c = jnp.transpose(bp1[...], (1, 0))
    bp2c = jnp.transpose(bp2[...], (1, 0))
    haT = jnp.maximum(
        jnp.dot(wa1T[...], aeT[...], preferred_element_type=jnp.float32)
        + ba1c, 0.0)
    oaT = jnp.maximum(
        jnp.dot(wa2T[...], haT, preferred_element_type=jnp.float32)
        + ba2c, 0.0)
    hpT = jnp.maximum(
        jnp.dot(wp1T[...], peT[...], preferred_element_type=jnp.float32)
        + bp1c, 0.0)
    opT = jnp.maximum(
        jnp.dot(wp2T[...], hpT, preferred_element_type=jnp.float32)
        + bp2c, 0.0)
    allT = jnp.concatenate([oaT, opT], axis=1)
    out_ref[...] = allT.reshape(8, 128)


_table_call = pl.pallas_call(
    _table_body,
    out_shape=jax.ShapeDtypeStruct((8, 128), jnp.float32),
)


@functools.partial(
    pl.kernel,
    out_type=jax.ShapeDtypeStruct((2 * NTOK,), jnp.float32),
    mesh=plsc.VectorSubcoreMesh(core_axis_name="c", subcore_axis_name="s"),
    scratch_types=[
        pltpu.VMEM((TPW,), jnp.int32),
        pltpu.VMEM((512,), jnp.float32),
        pltpu.VMEM((512,), jnp.float32),
        pltpu.VMEM((2 * TPW,), jnp.float32),
    ],
    compiler_params=pltpu.CompilerParams(needs_layout_passes=False),
)
def _gather_call(idx_hbm, table_hbm, out_hbm, idx_v, tab0_v, tab1_v, out_v):
    wid = lax.axis_index("s") * 2 + lax.axis_index("c")
    out_off = wid * TPW
    in_off = lax.rem(out_off + 4 * B, NTOK)
    pbase = jnp.where(wid < NW // 2, 0, 256).astype(jnp.int32)

    pltpu.sync_copy(table_hbm.at[pl.ds(0, 512)], tab0_v)
    pltpu.sync_copy(table_hbm.at[pl.ds(512, 512)], tab1_v)
    pltpu.sync_copy(idx_hbm.at[pl.ds(in_off, TPW)], idx_v)

    for b in range(TPW // 128):
        for s in range(8):
            fi = idx_v[pl.ds(b * 128 + s * L, L)] + pbase
            v0 = plsc.load_gather(tab0_v, [fi])
            v1 = plsc.load_gather(tab1_v, [fi])
            out_v[pl.ds(b * 256 + s * L, L)] = v0
            out_v[pl.ds(b * 256 + 128 + s * L, L)] = v1

    pltpu.sync_copy(out_v, out_hbm.at[pl.ds(2 * out_off, 2 * TPW)])


def kernel(inputs, addr_emb, pc_emb, Wa1, ba1, Wa2, ba2, Wp1, bp1, Wp2, bp2):
    table = _table_call(
        addr_emb.T, pc_emb.T,
        Wa1.T, ba1.reshape(1, 8), Wa2.T, ba2.reshape(1, 2),
        Wp1.T, bp1.reshape(1, 8), Wp2.T, bp2.reshape(1, 2))
    idx_flat = inputs.reshape(-1)
    out_flat = _gather_call(idx_flat, table.reshape(-1))
    return out_flat.reshape(NTOK // 128, 2, 128).transpose(0, 2, 1).reshape(NTOK, 2)


# speedup vs baseline: 1.0901x; 1.0901x over previous
import functools

import jax
import jax.numpy as jnp
from jax import lax
from jax.experimental import pallas as pl
from jax.experimental.pallas import tpu as pltpu
from jax.experimental.pallas import tpu_sc as plsc

B = 16384
NTOK = 8 * B
NW = 32
TPW = NTOK // NW
L = 16


def _table_body(aeT, peT, wa1T, ba1, wa2T, ba2, wp1T, bp1, wp2T, bp2, out_ref):
    ba1c = jnp.transpose(ba1[...], (1, 0))
    ba2c = jnp.transpose(ba2[...], (1, 0))
    bp1c = jnp.transpose(bp1[...], (1, 0))
    bp2c = jnp.transpose(bp2[...], (1, 0))
    haT = jnp.maximum(
        jnp.dot(wa1T[...], aeT[...], preferred_element_type=jnp.float32)
        + ba1c, 0.0)
    oaT = jnp.maximum(
        jnp.dot(wa2T[...], haT, preferred_element_type=jnp.float32)
        + ba2c, 0.0)
    hpT = jnp.maximum(
        jnp.dot(wp1T[...], peT[...], preferred_element_type=jnp.float32)
        + bp1c, 0.0)
    opT = jnp.maximum(
        jnp.dot(wp2T[...], hpT, preferred_element_type=jnp.float32)
        + bp2c, 0.0)
    allT = jnp.concatenate([oaT, opT], axis=1)
    out_ref[...] = allT.reshape(8, 128)


_table_call = pl.pallas_call(
    _table_body,
    out_shape=jax.ShapeDtypeStruct((8, 128), jnp.float32),
)


@functools.partial(
    pl.kernel,
    out_type=jax.ShapeDtypeStruct((2 * NTOK,), jnp.float32),
    mesh=plsc.VectorSubcoreMesh(core_axis_name="c", subcore_axis_name="s"),
    scratch_types=[
        pltpu.VMEM((TPW,), jnp.int32),
        pltpu.VMEM((512,), jnp.float32),
        pltpu.VMEM((512,), jnp.float32),
        pltpu.VMEM((2 * TPW,), jnp.float32),
    ],
    compiler_params=pltpu.CompilerParams(needs_layout_passes=False),
)
def _gather_call(idx_hbm, table_hbm, out_hbm, idx_v, tab0_v, tab1_v, out_v):
    wid = lax.axis_index("s") * 2 + lax.axis_index("c")
    out_off = wid * TPW
    in_off = lax.rem(out_off + 4 * B, NTOK)
    pbase = jnp.where(wid < NW // 2, 0, 256).astype(jnp.int32)

    pltpu.sync_copy(table_hbm.at[pl.ds(0, 512)], tab0_v)
    pltpu.sync_copy(table_hbm.at[pl.ds(512, 512)], tab1_v)
    pltpu.sync_copy(idx_hbm.at[pl.ds(in_off, TPW)], idx_v)

    @plsc.parallel_loop(0, TPW // 128, unroll=4)
    def _block(b):
        for s in range(8):
            fi = idx_v[pl.ds(b * 128 + s * L, L)] + pbase
            v0 = plsc.load_gather(tab0_v, [fi])
            v1 = plsc.load_gather(tab1_v, [fi])
            out_v[pl.ds(b * 256 + s * L, L)] = v0
            out_v[pl.ds(b * 256 + 128 + s * L, L)] = v1

    pltpu.sync_copy(out_v, out_hbm.at[pl.ds(2 * out_off, 2 * TPW)])


def kernel(inputs, addr_emb, pc_emb, Wa1, ba1, Wa2, ba2, Wp1, bp1, Wp2, bp2):
    table = _table_call(
        addr_emb.T, pc_emb.T,
        Wa1.T, ba1.reshape(1, 8), Wa2.T, ba2.reshape(1, 2),
        Wp1.T, bp1.reshape(1, 8), Wp2.T, bp2.reshape(1, 2))
    idx_flat = inputs.reshape(-1)
    out_flat = _gather_call(idx_flat, table.reshape(-1))
    return out_flat.reshape(NTOK // 128, 2, 128).transpose(0, 2, 1).reshape(NTOK, 2)
